# baseline (speedup 1.0000x reference)
import functools

import jax
import jax.numpy as jnp
from jax import lax
from jax.experimental import pallas as pl
from jax.experimental.pallas import tpu as pltpu
from jax.experimental.pallas import tpu_sc as plsc

N = 50000
E = 800000

NS = 16
CHUNK = 128
QW = 8

NP = 50048
TRASH = NP - 8
ROWS_PER_TILE = NP // NS
ZROWS = ROWS_PER_TILE // 4

EPAD = 4096 * 196
NCHUNKS = EPAD // CHUNK
CH_AGG = NCHUNKS // NS
CH_DEG = CH_AGG // 2
GFETCH = 56

_mesh = plsc.VectorSubcoreMesh(
    core_axis_name="c", subcore_axis_name="s", num_cores=1, num_subcores=NS
)
_sc_params = pltpu.CompilerParams(use_tc_tiling_on_sc=False)

NBUF = 4


def _sage_body(deg_half, *refs):
    if deg_half is not None:
        (t0, t1, t2, t3, src_hbm, dst_hbm, zeros_hbm,
         ones_hbm, out_hbm, deg_hbm,
         cid_v, idx_v, dst_v, rows_v, acc_sh, sem_g, sem_s) = refs
    else:
        (t0, t1, t2, t3, src_hbm, dst_hbm, zeros_hbm,
         out_hbm,
         cid_v, idx_v, dst_v, rows_v, acc_sh, sem_g, sem_s) = refs
    tables = (t0, t1, t2, t3)
    s = lax.axis_index("s")
    row0 = s * ROWS_PER_TILE
    rows = pl.ds(row0, ROWS_PER_TILE)
    drain_src = zeros_hbm.at[pl.ds(0, CHUNK)]

    def zero_acc():
        for z in range(4):
            pltpu.sync_copy(zeros_hbm,
                            acc_sh.at[pl.ds(row0 + z * ZROWS, ZROWS)])

    lane = jnp.arange(16, dtype=jnp.int32)
    for k in range((CH_AGG + 15) // 16):
        base = min(k * 16, CH_AGG - 16)
        cid_v[pl.ds(base, 16)] = s * CH_AGG + base + lane
    for g in range(CH_AGG // GFETCH):
        sl = pl.ds(g * GFETCH, GFETCH)
        pltpu.async_copy(src_hbm.at[cid_v.at[sl]], idx_v.at[sl],
                         sem_g.at[0]).wait()
        pltpu.async_copy(dst_hbm.at[cid_v.at[sl]], dst_v.at[sl],
                         sem_g.at[0]).wait()

    def g_start(p, j, b):
        pltpu.async_copy(tables[p].at[idx_v.at[j]], rows_v.at[b], sem_g.at[b])

    def g_wait(b):
        pltpu.make_async_copy(drain_src, rows_v.at[b], sem_g.at[b]).wait()

    def s_start(j, b):
        pltpu.async_copy(rows_v.at[b], acc_sh.at[dst_v.at[j]],
                         sem_s, add=True)

    def s_wait():
        pltpu.make_async_copy(drain_src, rows_v.at[0], sem_s).wait()

    for p in range(4):
        zero_acc()
        plsc.subcore_barrier()

        for b in range(NBUF):
            g_start(p, b, b)

        @pl.loop(0, CH_AGG, step=NBUF)
        def _(jj):
            for b in range(NBUF):
                g_wait(b)
                s_start(jj + b, b)
            for b in range(NBUF):
                s_wait()
            for b in range(NBUF):
                @pl.when(jj + NBUF + b < CH_AGG)
                def _():
                    g_start(p, jj + NBUF + b, b)

        plsc.subcore_barrier()
        pltpu.sync_copy(acc_sh.at[rows], out_hbm.at[p, rows])

    if deg_half is not None:
        pltpu.sync_copy(ones_hbm, rows_v.at[0])
        zero_acc()
        plsc.subcore_barrier()

        @pl.loop(deg_half * CH_DEG, (deg_half + 1) * CH_DEG, step=4)
        def _(jj):
            for b in range(4):
                pltpu.async_copy(rows_v.at[0], acc_sh.at[dst_v.at[jj + b]],
                                 sem_s, add=True)
            for b in range(4):
                s_wait()

        plsc.subcore_barrier()
        pltpu.sync_copy(acc_sh.at[rows], deg_hbm.at[rows])


_scratch = [
    pltpu.VMEM((CH_AGG,), jnp.int32),
    pltpu.VMEM((CH_AGG, CHUNK), jnp.int32),
    pltpu.VMEM((CH_AGG, CHUNK), jnp.int32),
    pltpu.VMEM((NBUF, CHUNK, QW), jnp.float32),
    pltpu.VMEM_SHARED((NP, QW), jnp.float32),
    pltpu.SemaphoreType.DMA((NBUF,)),
    pltpu.SemaphoreType.DMA,
]

_out_deg = [
    pltpu.HBM((4, NP, QW), jnp.float32),
    pltpu.HBM((NP, QW), jnp.float32),
]

_agg0A_call = pl.kernel(
    functools.partial(_sage_body, 0),
    out_type=_out_deg, mesh=_mesh,
    compiler_params=_sc_params, scratch_types=_scratch,
)

_agg0B_call = pl.kernel(
    functools.partial(_sage_body, 1),
    out_type=_out_deg, mesh=_mesh,
    compiler_params=_sc_params, scratch_types=_scratch,
)

_agg1_call = pl.kernel(
    functools.partial(_sage_body, None),
    out_type=pltpu.HBM((4, NP, QW), jnp.float32),
    mesh=_mesh,
    compiler_params=_sc_params, scratch_types=_scratch,
)


ROWS_TC = 2000


def _lin_body(qa, qb, x, d0, d1, wl, wr, b, o, *, final):
    agg = jnp.concatenate([qa[i] for i in range(4)]
                          + [qb[i] for i in range(4)], axis=1)
    deg = d0[:, 0:1] + d1[:, 0:1]
    inv = 1.0 / jnp.maximum(deg, 1.0)
    h = (jnp.dot(agg * inv, wl[...], preferred_element_type=jnp.float32)
         + jnp.dot(x[...], wr[...], preferred_element_type=jnp.float32)
         + b[...])
    if final:
        nrm = jnp.sqrt(jnp.sum(h * h, axis=1, keepdims=True))
        o[...] = h / jnp.maximum(nrm, 1e-12)
    else:
        o[...] = jnp.maximum(h, 0.0)


def _lin_call(qa, qb, d0, d1, x, wl, wr, b, final):
    grid = N // ROWS_TC
    body = functools.partial(_lin_body, final=final)
    qspec = pl.BlockSpec((4, ROWS_TC, QW), lambda i: (0, i, 0))
    dspec = pl.BlockSpec((ROWS_TC, QW), lambda i: (i, 0))
    return pl.pallas_call(
        body,
        grid=(grid,),
        in_specs=[
            qspec, qspec,
            pl.BlockSpec((ROWS_TC, 64), lambda i: (i, 0)),
            dspec, dspec,
            pl.BlockSpec((64, 64), lambda i: (0, 0)),
            pl.BlockSpec((64, 64), lambda i: (0, 0)),
            pl.BlockSpec((1, 64), lambda i: (0, 0)),
        ],
        out_specs=pl.BlockSpec((ROWS_TC, 64), lambda i: (i, 0)),
        out_shape=jax.ShapeDtypeStruct((N, 64), jnp.float32),
    )(qa, qb, x, d0, d1, wl, wr, b)


def kernel(edge_index, emb, Wl0, Wr0, b0, Wl1, Wr1, b1):
    src = edge_index[0]
    dst = edge_index[1]
    pad = EPAD - E
    src_t = jnp.concatenate([src, jnp.zeros((pad,), jnp.int32)])
    dst_t = jnp.concatenate([dst, jnp.full((pad,), TRASH, jnp.int32)])
    src_t = src_t.reshape(NCHUNKS, CHUNK)
    dst_t = dst_t.reshape(NCHUNKS, CHUNK)

    zeros = jnp.zeros((ZROWS, QW), jnp.float32)
    ones128 = jnp.ones((CHUNK, QW), jnp.float32)

    def tabsA(x):
        return tuple(x[:, 8 * p:8 * p + 8] for p in range(4))

    def tabsB(x):
        return tuple(x[:, 32 + 8 * p:40 + 8 * p] for p in range(4))

    agg0A, degA = _agg0A_call(*tabsA(emb), src_t, dst_t, zeros, ones128)
    agg0B, degB = _agg0B_call(*tabsB(emb), src_t, dst_t, zeros, ones128)
    h = _lin_call(agg0A[:, :N], agg0B[:, :N], degA[:N], degB[:N],
                  emb, Wl0, Wr0, b0.reshape(1, 64), final=False)

    agg1A = _agg1_call(*tabsA(h), src_t, dst_t, zeros)
    agg1B = _agg1_call(*tabsB(h), src_t, dst_t, zeros)
    out = _lin_call(agg1A[:, :N], agg1B[:, :N], degA[:N], degB[:N],
                    h, Wl1, Wr1, b1.reshape(1, 64), final=True)
    return out


# speedup vs baseline: 1.4332x; 1.4332x over previous
import functools

import jax
import jax.numpy as jnp
from jax import lax
from jax.experimental import pallas as pl
from jax.experimental.pallas import tpu as pltpu
from jax.experimental.pallas import tpu_sc as plsc

N = 50000
E = 800000

NC = 2
NS = 16
CHUNK = 128
QW = 8

NP = 50048
TRASH = NP - 8
ROWS_PER_TILE = NP // NS
ZROWS = ROWS_PER_TILE // 4

EPAD = 4096 * 196
NCHUNKS = EPAD // CHUNK
CH_AGG = NCHUNKS // NS
CH_DEG = CH_AGG // NC
GFETCH = 56

_mesh = plsc.VectorSubcoreMesh(
    core_axis_name="c", subcore_axis_name="s", num_cores=NC, num_subcores=NS
)
_sc_params = pltpu.CompilerParams(use_tc_tiling_on_sc=False)


NBUF = 4


def _sage_body(with_deg, *refs):
    if with_deg:
        (t0, t1, t2, t3, src_hbm, dst_hbm, zeros_hbm,
         ones_hbm, out_hbm, deg_hbm,
         cid_v, idx_v, dst_v, rows_v, acc_sh, sem_g, sem_s) = refs
    else:
        (t0, t1, t2, t3, src_hbm, dst_hbm, zeros_hbm,
         out_hbm,
         cid_v, idx_v, dst_v, rows_v, acc_sh, sem_g, sem_s) = refs
    tables = (t0, t1, t2, t3)
    c = lax.axis_index("c")
    s = lax.axis_index("s")
    row0 = s * ROWS_PER_TILE
    rows = pl.ds(row0, ROWS_PER_TILE)
    drain_src = zeros_hbm.at[pl.ds(0, CHUNK)]

    def zero_acc():
        for z in range(4):
            pltpu.sync_copy(zeros_hbm,
                            acc_sh.at[pl.ds(row0 + z * ZROWS, ZROWS)])

    lane = jnp.arange(16, dtype=jnp.int32)
    for k in range((CH_AGG + 15) // 16):
        base = min(k * 16, CH_AGG - 16)
        cid_v[pl.ds(base, 16)] = s * CH_AGG + base + lane
    for g in range(CH_AGG // GFETCH):
        sl = pl.ds(g * GFETCH, GFETCH)
        pltpu.async_copy(src_hbm.at[cid_v.at[sl]], idx_v.at[sl],
                         sem_g.at[0]).wait()
        pltpu.async_copy(dst_hbm.at[cid_v.at[sl]], dst_v.at[sl],
                         sem_g.at[0]).wait()

    cn = c * N

    def bias(r, carry):
        for l in range(CHUNK // 16):
            sl = pl.ds(l * 16, 16)
            idx_v[r, sl] = idx_v[r, sl] + cn
        return carry

    lax.fori_loop(0, CH_AGG, bias, 0)

    def g_start(p, j, b):
        pltpu.async_copy(tables[p].at[idx_v.at[j]], rows_v.at[b], sem_g.at[b])

    def g_wait(b):
        pltpu.make_async_copy(drain_src, rows_v.at[b], sem_g.at[b]).wait()

    def s_start(j, b):
        pltpu.async_copy(rows_v.at[b], acc_sh.at[dst_v.at[j]],
                         sem_s, add=True)

    def s_wait():
        pltpu.make_async_copy(drain_src, rows_v.at[0], sem_s).wait()

    for p in range(4):
        zero_acc()
        plsc.subcore_barrier()

        for b in range(NBUF):
            g_start(p, b, b)

        @pl.loop(0, CH_AGG, step=NBUF)
        def _(jj):
            for b in range(NBUF):
                g_wait(b)
                s_start(jj + b, b)
            for b in range(NBUF):
                s_wait()
            for b in range(NBUF):
                @pl.when(jj + NBUF + b < CH_AGG)
                def _():
                    g_start(p, jj + NBUF + b, b)

        plsc.subcore_barrier()
        pltpu.sync_copy(acc_sh.at[rows], out_hbm.at[4 * c + p, rows])

    if with_deg:
        pltpu.sync_copy(ones_hbm, rows_v.at[0])
        zero_acc()
        plsc.subcore_barrier()

        @pl.loop(c * CH_DEG, (c + 1) * CH_DEG, step=4)
        def _(jj):
            for b in range(4):
                pltpu.async_copy(rows_v.at[0], acc_sh.at[dst_v.at[jj + b]],
                                 sem_s, add=True)
            for b in range(4):
                s_wait()

        plsc.subcore_barrier()
        pltpu.sync_copy(acc_sh.at[rows], deg_hbm.at[c, rows])


_scratch = [
    pltpu.VMEM((CH_AGG,), jnp.int32),
    pltpu.VMEM((CH_AGG, CHUNK), jnp.int32),
    pltpu.VMEM((CH_AGG, CHUNK), jnp.int32),
    pltpu.VMEM((NBUF, CHUNK, QW), jnp.float32),
    pltpu.VMEM_SHARED((NP, QW), jnp.float32),
    pltpu.SemaphoreType.DMA((NBUF,)),
    pltpu.SemaphoreType.DMA,
]

_agg0_call = pl.kernel(
    functools.partial(_sage_body, True),
    out_type=[
        pltpu.HBM((8, NP, QW), jnp.float32),
        pltpu.HBM((NC, NP, QW), jnp.float32),
    ],
    mesh=_mesh,
    compiler_params=_sc_params,
    scratch_types=_scratch,
)

_agg1_call = pl.kernel(
    functools.partial(_sage_body, False),
    out_type=pltpu.HBM((8, NP, QW), jnp.float32),
    mesh=_mesh,
    compiler_params=_sc_params,
    scratch_types=_scratch,
)


ROWS_TC = 2000


def _lin_body(q, x, d, wl, wr, b, o, *, final):
    agg = jnp.concatenate([q[i] for i in range(8)], axis=1)
    deg = d[0, :, 0:1] + d[1, :, 0:1]
    inv = 1.0 / jnp.maximum(deg, 1.0)
    h = (jnp.dot(agg * inv, wl[...], preferred_element_type=jnp.float32)
         + jnp.dot(x[...], wr[...], preferred_element_type=jnp.float32)
         + b[...])
    if final:
        nrm = jnp.sqrt(jnp.sum(h * h, axis=1, keepdims=True))
        o[...] = h / jnp.maximum(nrm, 1e-12)
    else:
        o[...] = jnp.maximum(h, 0.0)


def _lin_call(q, d, x, wl, wr, b, final):
    grid = N // ROWS_TC
    body = functools.partial(_lin_body, final=final)
    return pl.pallas_call(
        body,
        grid=(grid,),
        in_specs=[
            pl.BlockSpec((8, ROWS_TC, QW), lambda i: (0, i, 0)),
            pl.BlockSpec((ROWS_TC, 64), lambda i: (i, 0)),
            pl.BlockSpec((NC, ROWS_TC, QW), lambda i: (0, i, 0)),
            pl.BlockSpec((64, 64), lambda i: (0, 0)),
            pl.BlockSpec((64, 64), lambda i: (0, 0)),
            pl.BlockSpec((1, 64), lambda i: (0, 0)),
        ],
        out_specs=pl.BlockSpec((ROWS_TC, 64), lambda i: (i, 0)),
        out_shape=jax.ShapeDtypeStruct((N, 64), jnp.float32),
    )(q, x, d, wl, wr, b)


def kernel(edge_index, emb, Wl0, Wr0, b0, Wl1, Wr1, b1):
    src = edge_index[0]
    dst = edge_index[1]
    pad = EPAD - E
    src_t = jnp.concatenate([src, jnp.zeros((pad,), jnp.int32)])
    dst_t = jnp.concatenate([dst, jnp.full((pad,), TRASH, jnp.int32)])
    src_t = src_t.reshape(NCHUNKS, CHUNK)
    dst_t = dst_t.reshape(NCHUNKS, CHUNK)

    zeros = jnp.zeros((ZROWS, QW), jnp.float32)
    ones128 = jnp.ones((CHUNK, QW), jnp.float32)

    def tables(x):
        return tuple(
            jnp.concatenate([x[:, 8 * p:8 * p + 8],
                             x[:, 32 + 8 * p:40 + 8 * p]], axis=0)
            for p in range(4))

    agg0, deg = _agg0_call(*tables(emb), src_t, dst_t, zeros, ones128)
    h = _lin_call(agg0, deg, emb, Wl0, Wr0,
                  b0.reshape(1, 64), final=False)

    agg1 = _agg1_call(*tables(h), src_t, dst_t, zeros)
    out = _lin_call(agg1, deg, h, Wl1, Wr1,
                    b1.reshape(1, 64), final=True)
    return out


# speedup vs baseline: 1.4972x; 1.0447x over previous
import functools

import jax
import jax.numpy as jnp
from jax import lax
from jax.experimental import pallas as pl
from jax.experimental.pallas import tpu as pltpu
from jax.experimental.pallas import tpu_sc as plsc

N = 50000
E = 800000

NC = 2
NS = 16
CHUNK = 128
QW = 8

NP = 50048
TRASH = NP - 8
ROWS_PER_TILE = NP // NS
ZROWS = ROWS_PER_TILE // 4

EPAD = 4096 * 196
NCHUNKS = EPAD // CHUNK
CH_AGG = NCHUNKS // NS
CH_DEG = CH_AGG // NC
GFETCH = 56

_mesh = plsc.VectorSubcoreMesh(
    core_axis_name="c", subcore_axis_name="s", num_cores=NC, num_subcores=NS
)
_sc_params = pltpu.CompilerParams(use_tc_tiling_on_sc=False)


NBUF = 4


def _sage_body(with_deg, *refs):
    if with_deg:
        (t0, t1, t2, t3, src_hbm, dst_hbm, zeros_hbm,
         ones_hbm, out_hbm, deg_hbm,
         cid_v, idx_v, dst_v, rows_v, acc_sh, sem_g, sem_s) = refs
    else:
        (t0, t1, t2, t3, src_hbm, dst_hbm, zeros_hbm,
         out_hbm,
         cid_v, idx_v, dst_v, rows_v, acc_sh, sem_g, sem_s) = refs
    tables = (t0, t1, t2, t3)
    c = lax.axis_index("c")
    s = lax.axis_index("s")
    row0 = s * ROWS_PER_TILE
    rows = pl.ds(row0, ROWS_PER_TILE)
    drain_src = zeros_hbm.at[pl.ds(0, CHUNK)]

    def zero_acc():
        for z in range(4):
            pltpu.sync_copy(zeros_hbm,
                            acc_sh.at[pl.ds(row0 + z * ZROWS, ZROWS)])

    lane = jnp.arange(16, dtype=jnp.int32)
    for k in range((CH_AGG + 15) // 16):
        base = min(k * 16, CH_AGG - 16)
        cid_v[pl.ds(base, 16)] = s * CH_AGG + base + lane
    for g in range(CH_AGG // GFETCH):
        sl = pl.ds(g * GFETCH, GFETCH)
        pltpu.async_copy(src_hbm.at[cid_v.at[sl]], idx_v.at[sl],
                         sem_g.at[0]).wait()
        pltpu.async_copy(dst_hbm.at[cid_v.at[sl]], dst_v.at[sl],
                         sem_g.at[0]).wait()

    cn = c * N

    def bias(r, carry):
        for l in range(CHUNK // 16):
            sl = pl.ds(l * 16, 16)
            idx_v[r, sl] = idx_v[r, sl] + cn
        return carry

    lax.fori_loop(0, CH_AGG, bias, 0)

    def g_start(p, j, b):
        pltpu.async_copy(tables[p].at[idx_v.at[j]], rows_v.at[b], sem_g.at[b])

    def g_wait(b):
        pltpu.make_async_copy(drain_src, rows_v.at[b], sem_g.at[b]).wait()

    def s_start(j, b):
        pltpu.async_copy(rows_v.at[b], acc_sh.at[dst_v.at[j]],
                         sem_s.at[b], add=True)

    def s_wait(b):
        pltpu.make_async_copy(drain_src, rows_v.at[b], sem_s.at[b]).wait()

    for p in range(4):
        zero_acc()
        plsc.subcore_barrier()

        for b in range(NBUF):
            g_start(p, b, b)

        @pl.loop(0, CH_AGG, step=NBUF)
        def _(jj):
            for b in range(NBUF):
                g_wait(b)
                s_start(jj + b, b)
            for b in range(NBUF):
                s_wait(b)

                @pl.when(jj + NBUF + b < CH_AGG)
                def _():
                    g_start(p, jj + NBUF + b, b)

        plsc.subcore_barrier()
        pltpu.sync_copy(acc_sh.at[rows], out_hbm.at[4 * c + p, rows])

    if with_deg:
        pltpu.sync_copy(ones_hbm, rows_v.at[0])
        zero_acc()
        plsc.subcore_barrier()

        @pl.loop(c * CH_DEG, (c + 1) * CH_DEG, step=4)
        def _(jj):
            for b in range(4):
                pltpu.async_copy(rows_v.at[0], acc_sh.at[dst_v.at[jj + b]],
                                 sem_s.at[b], add=True)
            for b in range(4):
                s_wait(b)

        plsc.subcore_barrier()
        pltpu.sync_copy(acc_sh.at[rows], deg_hbm.at[c, rows])


_scratch = [
    pltpu.VMEM((CH_AGG,), jnp.int32),
    pltpu.VMEM((CH_AGG, CHUNK), jnp.int32),
    pltpu.VMEM((CH_AGG, CHUNK), jnp.int32),
    pltpu.VMEM((NBUF, CHUNK, QW), jnp.float32),
    pltpu.VMEM_SHARED((NP, QW), jnp.float32),
    pltpu.SemaphoreType.DMA((NBUF,)),
    pltpu.SemaphoreType.DMA((NBUF,)),
]

_agg0_call = pl.kernel(
    functools.partial(_sage_body, True),
    out_type=[
        pltpu.HBM((8, NP, QW), jnp.float32),
        pltpu.HBM((NC, NP, QW), jnp.float32),
    ],
    mesh=_mesh,
    compiler_params=_sc_params,
    scratch_types=_scratch,
)

_agg1_call = pl.kernel(
    functools.partial(_sage_body, False),
    out_type=pltpu.HBM((8, NP, QW), jnp.float32),
    mesh=_mesh,
    compiler_params=_sc_params,
    scratch_types=_scratch,
)


ROWS_TC = 2000


def _lin_body(q, x, d, wl, wr, b, o, *, final):
    agg = jnp.concatenate([q[i] for i in range(8)], axis=1)
    deg = d[0, :, 0:1] + d[1, :, 0:1]
    inv = 1.0 / jnp.maximum(deg, 1.0)
    h = (jnp.dot(agg * inv, wl[...], preferred_element_type=jnp.float32)
         + jnp.dot(x[...], wr[...], preferred_element_type=jnp.float32)
         + b[...])
    if final:
        nrm = jnp.sqrt(jnp.sum(h * h, axis=1, keepdims=True))
        o[...] = h / jnp.maximum(nrm, 1e-12)
    else:
        o[...] = jnp.maximum(h, 0.0)


def _lin_call(q, d, x, wl, wr, b, final):
    grid = N // ROWS_TC
    body = functools.partial(_lin_body, final=final)
    return pl.pallas_call(
        body,
        grid=(grid,),
        in_specs=[
            pl.BlockSpec((8, ROWS_TC, QW), lambda i: (0, i, 0)),
            pl.BlockSpec((ROWS_TC, 64), lambda i: (i, 0)),
            pl.BlockSpec((NC, ROWS_TC, QW), lambda i: (0, i, 0)),
            pl.BlockSpec((64, 64), lambda i: (0, 0)),
            pl.BlockSpec((64, 64), lambda i: (0, 0)),
            pl.BlockSpec((1, 64), lambda i: (0, 0)),
        ],
        out_specs=pl.BlockSpec((ROWS_TC, 64), lambda i: (i, 0)),
        out_shape=jax.ShapeDtypeStruct((N, 64), jnp.float32),
    )(q, x, d, wl, wr, b)


def kernel(edge_index, emb, Wl0, Wr0, b0, Wl1, Wr1, b1):
    src = edge_index[0]
    dst = edge_index[1]
    pad = EPAD - E
    src_t = jnp.concatenate([src, jnp.zeros((pad,), jnp.int32)])
    dst_t = jnp.concatenate([dst, jnp.full((pad,), TRASH, jnp.int32)])
    src_t = src_t.reshape(NCHUNKS, CHUNK)
    dst_t = dst_t.reshape(NCHUNKS, CHUNK)

    zeros = jnp.zeros((ZROWS, QW), jnp.float32)
    ones128 = jnp.ones((CHUNK, QW), jnp.float32)

    def tables(x):
        return tuple(
            jnp.concatenate([x[:, 8 * p:8 * p + 8],
                             x[:, 32 + 8 * p:40 + 8 * p]], axis=0)
            for p in range(4))

    agg0, deg = _agg0_call(*tables(emb), src_t, dst_t, zeros, ones128)
    h = _lin_call(agg0, deg, emb, Wl0, Wr0,
                  b0.reshape(1, 64), final=False)

    agg1 = _agg1_call(*tables(h), src_t, dst_t, zeros)
    out = _lin_call(agg1, deg, h, Wl1, Wr1,
                    b1.reshape(1, 64), final=True)
    return out
